# Initial kernel scaffold; baseline (speedup 1.0000x reference)
import jax
import jax.numpy as jnp
from jax.experimental import pallas as pl


def kernel(z, W):
    raise NotImplementedError("write your pallas kernel here")


# speedup vs baseline: 1.1759x; 1.1759x over previous
import functools

import jax
import jax.numpy as jnp
from jax import lax
from jax.experimental import pallas as pl
from jax.experimental.pallas import tpu as pltpu
from jax.experimental.pallas import tpu_sc as plsc

_K = 8192
_D = 256
_B = 16384
_BT = 256
_NB = _B // _BT


def _dist_argmin_body(z_ref, w_ref, zz_ref, ww_ref, idx_ref, loss_ref, acc_ref):
    i = pl.program_id(0)
    mm = lax.dot_general(
        z_ref[...], w_ref[...], (((1,), (1,)), ((), ())),
        preferred_element_type=jnp.float32)
    dists = (zz_ref[...] + ww_ref[...]) - 2.0 * mm
    m = jnp.min(dists, axis=-1, keepdims=True)
    iota = lax.broadcasted_iota(jnp.int32, (_BT, _K), 1)
    idx = jnp.min(jnp.where(dists == m, iota, _K), axis=-1)
    idx_ref[0, 0, :] = idx

    @pl.when(i == 0)
    def _init():
        acc_ref[0] = 0.0

    acc_ref[0] += jnp.sum(m)

    @pl.when(i == _NB - 1)
    def _fini():
        s = acc_ref[0] / float(_B * _D)
        loss_ref[...] = jnp.full((1, 1), s + 0.25 * s, jnp.float32)


def _dist_argmin(z, W, zz, ww):
    return pl.pallas_call(
        _dist_argmin_body,
        grid=(_NB,),
        in_specs=[
            pl.BlockSpec((_BT, _D), lambda i: (i, 0)),
            pl.BlockSpec((_K, _D), lambda i: (0, 0)),
            pl.BlockSpec((_BT, 1), lambda i: (i, 0)),
            pl.BlockSpec((1, _K), lambda i: (0, 0)),
        ],
        out_specs=[
            pl.BlockSpec((1, 1, _BT), lambda i: (i, 0, 0)),
            pl.BlockSpec((1, 1), lambda i: (0, 0)),
        ],
        out_shape=[
            jax.ShapeDtypeStruct((_NB, 1, _BT), jnp.int32),
            jax.ShapeDtypeStruct((1, 1), jnp.float32),
        ],
        scratch_shapes=[pltpu.SMEM((1,), jnp.float32)],
    )(z, W, zz, ww)


_NC, _NS = 2, 16
_NW = _NC * _NS
_BPW = _B // _NW
_CH = 256
_NCH = _BPW // _CH


def _gather_body(w_hbm, idx_hbm, out_hbm, idx_v, rows_v, sem):
    wid = lax.axis_index("s") * _NC + lax.axis_index("c")
    base = wid * _BPW
    for c in range(_NCH):
        off = base + c * _CH
        pltpu.sync_copy(idx_hbm.at[pl.ds(off, _CH)], idx_v)
        pltpu.async_copy(w_hbm.at[idx_v], rows_v, sem).wait()
        pltpu.sync_copy(rows_v, out_hbm.at[pl.ds(off, _CH)])


@functools.cache
def _make_gather():
    return pl.kernel(
        _gather_body,
        out_type=jax.ShapeDtypeStruct((_B, _D), jnp.float32),
        mesh=plsc.VectorSubcoreMesh(core_axis_name="c", subcore_axis_name="s"),
        scratch_types=[
            pltpu.VMEM((_CH,), jnp.int32),
            pltpu.VMEM((_CH, _D), jnp.float32),
            pltpu.SemaphoreType.DMA,
        ],
    )


def kernel(z, W):
    zz = jnp.sum(z * z, axis=-1, keepdims=True)
    ww = jnp.sum(W * W, axis=-1)
    idx3, loss = _dist_argmin(z, W, zz, ww.reshape(1, _K))
    indices = idx3.reshape(_B)
    z_q = _make_gather()(W, indices)
    z_q_st = z + (z_q - z)
    return (z_q_st, indices, loss[0, 0])


# speedup vs baseline: 1.2132x; 1.0317x over previous
import functools

import jax
import jax.numpy as jnp
from jax import lax
from jax.experimental import pallas as pl
from jax.experimental.pallas import tpu as pltpu
from jax.experimental.pallas import tpu_sc as plsc

_K = 8192
_D = 256
_B = 16384
_BT = 512
_NB = _B // _BT
_CW = 2048
_NCW = _K // _CW


def _dist_argmin_body(z_ref, w_ref, zz_ref, ww_ref, it_ref, idx_ref, loss_ref,
                      d_ref, acc_ref):
    i = pl.program_id(0)
    z2 = 2.0 * z_ref[...]
    zzr = zz_ref[...]
    m = None
    for c in range(_NCW):
        ks = pl.ds(c * _CW, _CW)
        mm2 = lax.dot_general(
            z2, w_ref[ks, :], (((1,), (1,)), ((), ())),
            preferred_element_type=jnp.float32)
        dc = (zzr + ww_ref[:, ks]) - mm2
        d_ref[:, ks] = dc
        mc = jnp.min(dc, axis=-1, keepdims=True)
        m = mc if m is None else jnp.minimum(m, mc)
    idxf = None
    for c in range(_NCW):
        ks = pl.ds(c * _CW, _CW)
        cand = jnp.where(d_ref[:, ks] == m, it_ref[:, ks], float(_K))
        ic = jnp.min(cand, axis=-1, keepdims=True)
        idxf = ic if idxf is None else jnp.minimum(idxf, ic)
    idx_ref[0, 0, :] = idxf[:, 0].astype(jnp.int32)

    @pl.when(i == 0)
    def _init():
        acc_ref[0] = 0.0

    acc_ref[0] += jnp.sum(m)

    @pl.when(i == _NB - 1)
    def _fini():
        s = acc_ref[0] / float(_B * _D)
        loss_ref[...] = jnp.full((1, 1), s + 0.25 * s, jnp.float32)


def _dist_argmin(z, W, zz, ww, it):
    return pl.pallas_call(
        _dist_argmin_body,
        grid=(_NB,),
        in_specs=[
            pl.BlockSpec((_BT, _D), lambda i: (i, 0)),
            pl.BlockSpec((_K, _D), lambda i: (0, 0)),
            pl.BlockSpec((_BT, 1), lambda i: (i, 0)),
            pl.BlockSpec((1, _K), lambda i: (0, 0)),
            pl.BlockSpec((1, _K), lambda i: (0, 0)),
        ],
        out_specs=[
            pl.BlockSpec((1, 1, _BT), lambda i: (i, 0, 0)),
            pl.BlockSpec((1, 1), lambda i: (0, 0)),
        ],
        out_shape=[
            jax.ShapeDtypeStruct((_NB, 1, _BT), jnp.int32),
            jax.ShapeDtypeStruct((1, 1), jnp.float32),
        ],
        scratch_shapes=[
            pltpu.VMEM((_BT, _K), jnp.float32),
            pltpu.SMEM((1,), jnp.float32),
        ],
    )(z, W, zz, ww, it)


_NC, _NS = 2, 16
_NW = _NC * _NS
_BPW = _B // _NW
_CH = 256
_NCH = _BPW // _CH


def _gather_body(w_hbm, idx_hbm, out_hbm, idx_v, rows_v, sem):
    wid = lax.axis_index("s") * _NC + lax.axis_index("c")
    base = wid * _BPW
    for c in range(_NCH):
        off = base + c * _CH
        pltpu.sync_copy(idx_hbm.at[pl.ds(off, _CH)], idx_v)
        pltpu.async_copy(w_hbm.at[idx_v], rows_v, sem).wait()
        pltpu.sync_copy(rows_v, out_hbm.at[pl.ds(off, _CH)])


@functools.cache
def _make_gather():
    return pl.kernel(
        _gather_body,
        out_type=jax.ShapeDtypeStruct((_B, _D), jnp.float32),
        mesh=plsc.VectorSubcoreMesh(core_axis_name="c", subcore_axis_name="s"),
        scratch_types=[
            pltpu.VMEM((_CH,), jnp.int32),
            pltpu.VMEM((_CH, _D), jnp.float32),
            pltpu.SemaphoreType.DMA,
        ],
    )


def kernel(z, W):
    zz = jnp.sum(z * z, axis=-1, keepdims=True)
    ww = jnp.sum(W * W, axis=-1)
    it = jnp.arange(_K, dtype=jnp.float32).reshape(1, _K)
    idx3, loss = _dist_argmin(z, W, zz, ww.reshape(1, _K), it)
    indices = idx3.reshape(_B)
    z_q = _make_gather()(W, indices)
    z_q_st = z + (z_q - z)
    return (z_q_st, indices, loss[0, 0])


# speedup vs baseline: 1.2902x; 1.0635x over previous
import functools

import jax
import jax.numpy as jnp
from jax import lax
from jax.experimental import pallas as pl
from jax.experimental.pallas import tpu as pltpu
from jax.experimental.pallas import tpu_sc as plsc

_K = 8192
_D = 256
_B = 16384
_BT = 512
_NB = _B // _BT
_CW = 2048
_NCW = _K // _CW


def _dist_argmin_body(z_ref, w_ref, zz_ref, ww_ref, it_ref, idx_ref, loss_ref):
    z2 = 2.0 * z_ref[...]
    zzr = zz_ref[...]
    m = None
    idxf = None
    for c in range(_NCW):
        ks = pl.ds(c * _CW, _CW)
        mm2 = lax.dot_general(
            z2, w_ref[ks, :], (((1,), (1,)), ((), ())),
            preferred_element_type=jnp.float32)
        dc = (zzr + ww_ref[:, ks]) - mm2
        mc = jnp.min(dc, axis=-1, keepdims=True)
        cand = jnp.where(dc == mc, it_ref[:, ks], float(_K))
        ic = jnp.min(cand, axis=-1, keepdims=True)
        if m is None:
            m, idxf = mc, ic
        else:
            idxf = jnp.where(mc < m, ic, idxf)
            m = jnp.minimum(m, mc)
    idx_ref[0, 0, :] = idxf[:, 0].astype(jnp.int32)
    loss_ref[...] = jnp.full((1, 1, 1), jnp.sum(m), jnp.float32)


def _dist_argmin(z, W, zz, ww, it):
    return pl.pallas_call(
        _dist_argmin_body,
        grid=(_NB,),
        in_specs=[
            pl.BlockSpec((_BT, _D), lambda i: (i, 0)),
            pl.BlockSpec((_K, _D), lambda i: (0, 0)),
            pl.BlockSpec((_BT, 1), lambda i: (i, 0)),
            pl.BlockSpec((1, _K), lambda i: (0, 0)),
            pl.BlockSpec((1, _K), lambda i: (0, 0)),
        ],
        out_specs=[
            pl.BlockSpec((1, 1, _BT), lambda i: (i, 0, 0)),
            pl.BlockSpec((1, 1, 1), lambda i: (i, 0, 0)),
        ],
        out_shape=[
            jax.ShapeDtypeStruct((_NB, 1, _BT), jnp.int32),
            jax.ShapeDtypeStruct((_NB, 1, 1), jnp.float32),
        ],
        compiler_params=pltpu.CompilerParams(
            dimension_semantics=("parallel",)),
    )(z, W, zz, ww, it)


_NC, _NS = 2, 16
_NW = _NC * _NS
_BPW = _B // _NW
_CH = 256
_NCH = _BPW // _CH


def _gather_body(w_hbm, idx_hbm, out_hbm, idx_v, rows_v, sem):
    wid = lax.axis_index("s") * _NC + lax.axis_index("c")
    base = wid * _BPW
    for c in range(_NCH):
        off = base + c * _CH
        pltpu.sync_copy(idx_hbm.at[pl.ds(off, _CH)], idx_v)
        pltpu.async_copy(w_hbm.at[idx_v], rows_v, sem).wait()
        pltpu.sync_copy(rows_v, out_hbm.at[pl.ds(off, _CH)])


@functools.cache
def _make_gather():
    return pl.kernel(
        _gather_body,
        out_type=jax.ShapeDtypeStruct((_B, _D), jnp.float32),
        mesh=plsc.VectorSubcoreMesh(core_axis_name="c", subcore_axis_name="s"),
        scratch_types=[
            pltpu.VMEM((_CH,), jnp.int32),
            pltpu.VMEM((_CH, _D), jnp.float32),
            pltpu.SemaphoreType.DMA,
        ],
    )


def kernel(z, W):
    zz = jnp.sum(z * z, axis=-1, keepdims=True)
    ww = jnp.sum(W * W, axis=-1)
    it = jnp.arange(_K, dtype=jnp.float32).reshape(1, _K)
    idx3, loss_parts = _dist_argmin(z, W, zz, ww.reshape(1, _K), it)
    indices = idx3.reshape(_B)
    z_q = _make_gather()(W, indices)
    s = jnp.sum(loss_parts) / float(_B * _D)
    vq_loss = s + 0.25 * s
    return (z_q, indices, vq_loss)


# speedup vs baseline: 1.3919x; 1.0788x over previous
import functools

import jax
import jax.numpy as jnp
from jax import lax
from jax.experimental import pallas as pl
from jax.experimental.pallas import tpu as pltpu
from jax.experimental.pallas import tpu_sc as plsc

_K = 8192
_D = 256
_B = 16384
_BT = 512
_NB = _B // _BT
_CW = 2048
_NCW = _K // _CW


def _dist_argmin_body(z_ref, w_ref, idx_ref, loss_ref, ww_ref, acc_ref):
    i = pl.program_id(0)

    @pl.when(i == 0)
    def _norms():
        wb = w_ref[...]
        ww_ref[0, :] = jnp.sum(wb * wb, axis=-1)
        acc_ref[0] = 0.0

    zb = z_ref[...]
    zzr = jnp.sum(zb * zb, axis=-1, keepdims=True)
    z2 = 2.0 * zb
    m = None
    idxf = None
    for c in range(_NCW):
        ks = pl.ds(c * _CW, _CW)
        mm2 = lax.dot_general(
            z2, w_ref[ks, :], (((1,), (1,)), ((), ())),
            preferred_element_type=jnp.float32)
        dc = (zzr + ww_ref[:, ks]) - mm2
        mc = jnp.min(dc, axis=-1, keepdims=True)
        itc = (lax.broadcasted_iota(jnp.int32, (1, _CW), 1)
               .astype(jnp.float32) + float(c * _CW))
        cand = jnp.where(dc == mc, itc, float(_K))
        ic = jnp.min(cand, axis=-1, keepdims=True)
        if m is None:
            m, idxf = mc, ic
        else:
            idxf = jnp.where(mc < m, ic, idxf)
            m = jnp.minimum(m, mc)
    idx_ref[0, 0, :] = idxf[:, 0].astype(jnp.int32)

    acc_ref[0] += jnp.sum(m)

    @pl.when(i == _NB - 1)
    def _fini():
        s = acc_ref[0] / float(_B * _D)
        loss_ref[...] = jnp.full((1, 1), s + 0.25 * s, jnp.float32)


def _dist_argmin(z, W):
    return pl.pallas_call(
        _dist_argmin_body,
        grid=(_NB,),
        in_specs=[
            pl.BlockSpec((_BT, _D), lambda i: (i, 0)),
            pl.BlockSpec((_K, _D), lambda i: (0, 0)),
        ],
        out_specs=[
            pl.BlockSpec((1, 1, _BT), lambda i: (i, 0, 0)),
            pl.BlockSpec((1, 1), lambda i: (0, 0)),
        ],
        out_shape=[
            jax.ShapeDtypeStruct((_NB, 1, _BT), jnp.int32),
            jax.ShapeDtypeStruct((1, 1), jnp.float32),
        ],
        scratch_shapes=[
            pltpu.VMEM((1, _K), jnp.float32),
            pltpu.SMEM((1,), jnp.float32),
        ],
    )(z, W)


_NC, _NS = 2, 16
_NW = _NC * _NS
_BPW = _B // _NW
_CH = 256
_NCH = _BPW // _CH


def _gather_body(w_hbm, idx_hbm, out_hbm, idx_v, rows_v, sem):
    wid = lax.axis_index("s") * _NC + lax.axis_index("c")
    base = wid * _BPW
    for c in range(_NCH):
        off = base + c * _CH
        pltpu.sync_copy(idx_hbm.at[pl.ds(off, _CH)], idx_v)
        pltpu.async_copy(w_hbm.at[idx_v], rows_v, sem).wait()
        pltpu.sync_copy(rows_v, out_hbm.at[pl.ds(off, _CH)])


@functools.cache
def _make_gather():
    return pl.kernel(
        _gather_body,
        out_type=jax.ShapeDtypeStruct((_B, _D), jnp.float32),
        mesh=plsc.VectorSubcoreMesh(core_axis_name="c", subcore_axis_name="s"),
        scratch_types=[
            pltpu.VMEM((_CH,), jnp.int32),
            pltpu.VMEM((_CH, _D), jnp.float32),
            pltpu.SemaphoreType.DMA,
        ],
    )


def kernel(z, W):
    idx3, loss = _dist_argmin(z, W)
    indices = idx3.reshape(_B)
    z_q = _make_gather()(W, indices)
    return (z_q, indices, loss[0, 0])
